# baseline (speedup 1.0000x reference)
import functools

import jax
import jax.numpy as jnp
from jax.experimental import pallas as pl
from jax.experimental.pallas import tpu as pltpu

B_SZ, I, F, H = 8192, 16, 128, 32
BB = 256
GW = 3 * F + H


def _weights_body(A_ref, B_ref, nA_ref, G_ref):
    dot = functools.partial(jax.lax.dot_general,
                            preferred_element_type=jnp.float32)
    for i in range(I):
        A_i = A_ref[i]
        B_i = B_ref[i]
        inv = jax.lax.rsqrt(jnp.sum(A_i * A_i, axis=0, keepdims=True))
        nA = A_i * inv
        M = dot(B_i, B_i, (((1,), (1,)), ((), ())))
        K1 = dot(nA, M, (((1,), (0,)), ((), ())))
        K2 = dot(K1, nA, (((1,), (1,)), ((), ())))
        K0 = dot(nA, B_i, (((1,), (0,)), ((), ())))
        nA_ref[i] = nA
        G_ref[i, :, 0:F] = nA
        G_ref[i, :, F:2 * F] = K1
        G_ref[i, :, 2 * F:3 * F] = K2
        G_ref[i, :, 3 * F:] = K0


def _fused_body(feat_ref, G_ref, bf_ref,
                out_ref, h0_ref, h1_ref, hid_ref, pre_ref,
                ybuf, sem):
    dot = functools.partial(jax.lax.dot_general,
                            preferred_element_type=jnp.float32)
    pending = []
    for i in range(I):
        x = feat_ref[:, i, :]
        Y = dot(x, G_ref[i], (((1,), (0,)), ((), ())))
        slot = i % 2
        if i >= 2:
            for cp in pending[i - 2]:
                cp.wait()
        ybuf[slot, :, 0:2 * F] = Y[:, 0:2 * F]
        ybuf[slot, :, 2 * F:3 * F] = Y[:, 2 * F:3 * F] + bf_ref[i][None, :]
        hid_ref[:, i, :] = Y[:, 3 * F:]
        copies = (
            pltpu.make_async_copy(ybuf.at[slot, :, 0:F],
                                  h0_ref.at[:, i, :], sem),
            pltpu.make_async_copy(ybuf.at[slot, :, F:2 * F],
                                  h1_ref.at[:, i, :], sem),
            pltpu.make_async_copy(ybuf.at[slot, :, 2 * F:3 * F],
                                  pre_ref.at[:, i, :], sem),
        )
        for cp in copies:
            cp.start()
        pending.append(copies)
    for cps in pending[I - 2:]:
        for cp in cps:
            cp.wait()
    out_ref[...] = jnp.maximum(pre_ref[...], 0.0)


def kernel(features, A, B, b_final):
    nA, G = pl.pallas_call(
        _weights_body,
        out_shape=(jax.ShapeDtypeStruct((I, F, F), jnp.float32),
                   jax.ShapeDtypeStruct((I, F, GW), jnp.float32)),
    )(A, B)

    nbb = B_SZ // BB
    batch_spec = pl.BlockSpec((BB, I, F), lambda b: (b, 0, 0))

    in_specs = (
        batch_spec,
        pl.BlockSpec((I, F, GW), lambda b: (0, 0, 0)),
        pl.BlockSpec((I, F), lambda b: (0, 0)),
    )
    out_specs = (
        batch_spec,
        batch_spec,
        batch_spec,
        pl.BlockSpec((BB, I, H), lambda b: (b, 0, 0)),
        batch_spec,
    )
    out_shape = (
        jax.ShapeDtypeStruct((B_SZ, I, F), jnp.float32),
        jax.ShapeDtypeStruct((B_SZ, I, F), jnp.float32),
        jax.ShapeDtypeStruct((B_SZ, I, F), jnp.float32),
        jax.ShapeDtypeStruct((B_SZ, I, H), jnp.float32),
        jax.ShapeDtypeStruct((B_SZ, I, F), jnp.float32),
    )

    out, h0, h1, hid, pre = pl.pallas_call(
        _fused_body,
        grid=(nbb,),
        in_specs=in_specs,
        out_specs=out_specs,
        out_shape=out_shape,
        compiler_params=pltpu.CompilerParams(
            dimension_semantics=("arbitrary",),
        ),
        scratch_shapes=[pltpu.VMEM((2, BB, 3 * F), jnp.float32),
                        pltpu.SemaphoreType.DMA],
    )(features, G, b_final)
    return (out, h0, h1, hid, pre, nA)


# speedup vs baseline: 1.4810x; 1.4810x over previous
import functools

import jax
import jax.numpy as jnp
from jax.experimental import pallas as pl
from jax.experimental.pallas import tpu as pltpu

B_SZ, I, F, H = 8192, 16, 128, 32
BB = 256
GW = 3 * F + H


def _weights_body(A_ref, B_ref, nA_ref, G_ref):
    dot = functools.partial(jax.lax.dot_general,
                            preferred_element_type=jnp.float32)
    for i in range(I):
        A_i = A_ref[i]
        B_i = B_ref[i]
        inv = jax.lax.rsqrt(jnp.sum(A_i * A_i, axis=0, keepdims=True))
        nA = A_i * inv
        M = dot(B_i, B_i, (((1,), (1,)), ((), ())))
        K1 = dot(nA, M, (((1,), (0,)), ((), ())))
        K2 = dot(K1, nA, (((1,), (1,)), ((), ())))
        K0 = dot(nA, B_i, (((1,), (0,)), ((), ())))
        nA_ref[i] = nA
        G_ref[i, :, 0:F] = nA
        G_ref[i, :, F:2 * F] = K1
        G_ref[i, :, 2 * F:3 * F] = K2
        G_ref[i, :, 3 * F:] = K0


def _fused_body(feat_ref, G_ref, bf_ref,
                out_ref, h0_ref, h1_ref, hid_ref, pre_ref):
    dot = functools.partial(jax.lax.dot_general,
                            preferred_element_type=jnp.float32)
    for i in range(I):
        x = feat_ref[:, i, :]
        Y = dot(x, G_ref[i], (((1,), (0,)), ((), ())))
        pre = Y[:, 2 * F:3 * F] + bf_ref[i][None, :]
        h0_ref[:, i, :] = Y[:, 0:F]
        h1_ref[:, i, :] = Y[:, F:2 * F]
        hid_ref[:, i, :] = Y[:, 3 * F:]
        pre_ref[:, i, :] = pre
    out_ref[...] = jnp.maximum(pre_ref[...], 0.0)


def kernel(features, A, B, b_final):
    nA, G = pl.pallas_call(
        _weights_body,
        out_shape=(jax.ShapeDtypeStruct((I, F, F), jnp.float32),
                   jax.ShapeDtypeStruct((I, F, GW), jnp.float32)),
    )(A, B)

    nbb = B_SZ // BB
    batch_spec = pl.BlockSpec((BB, I, F), lambda b: (b, 0, 0))

    in_specs = (
        batch_spec,
        pl.BlockSpec((I, F, GW), lambda b: (0, 0, 0)),
        pl.BlockSpec((I, F), lambda b: (0, 0)),
    )
    out_specs = (
        batch_spec,
        batch_spec,
        batch_spec,
        pl.BlockSpec((BB, I, H), lambda b: (b, 0, 0)),
        batch_spec,
    )
    out_shape = (
        jax.ShapeDtypeStruct((B_SZ, I, F), jnp.float32),
        jax.ShapeDtypeStruct((B_SZ, I, F), jnp.float32),
        jax.ShapeDtypeStruct((B_SZ, I, F), jnp.float32),
        jax.ShapeDtypeStruct((B_SZ, I, H), jnp.float32),
        jax.ShapeDtypeStruct((B_SZ, I, F), jnp.float32),
    )

    out, h0, h1, hid, pre = pl.pallas_call(
        _fused_body,
        grid=(nbb,),
        in_specs=in_specs,
        out_specs=out_specs,
        out_shape=out_shape,
        compiler_params=pltpu.CompilerParams(
            dimension_semantics=("arbitrary",),
        ),
    )(features, G, b_final)
    return (out, h0, h1, hid, pre, nA)
